# baseline (speedup 1.0000x reference)
import jax
import jax.numpy as jnp
from jax.experimental import pallas as pl
from jax.experimental.pallas import tpu as pltpu


def kernel(tokens, positional_embedding_weights):
    batch_size, seq_len = tokens.shape
    pos = positional_embedding_weights[:seq_len]
    S, D = pos.shape
    CH = 4096
    n_chunks = S // CH
    NBUF = 2
    AHEAD = 1

    def body(in_hbm, out_hbm, *refs):
        bufs = refs[:NBUF]
        rsems = refs[NBUF : 2 * NBUF]
        wsems = refs[2 * NBUF :]

        def read(c):
            s = c % NBUF
            cp = pltpu.make_async_copy(in_hbm.at[pl.ds(c * CH, CH)], bufs[s], rsems[s])
            cp.start()
            return cp

        def write(c):
            s = c % NBUF
            half = CH // 2
            cps = []
            for b in range(batch_size):
                for h2 in range(2):
                    cp = pltpu.make_async_copy(
                        bufs[s].at[pl.ds(h2 * half, half)],
                        out_hbm.at[b, pl.ds(c * CH + h2 * half, half)],
                        wsems[s * batch_size + b],
                    )
                    cp.start()
                    cps.append(cp)
            return cps

        pending_writes = [None] * NBUF
        pending_reads = [None] * n_chunks
        for c in range(min(AHEAD + 1, n_chunks)):
            pending_reads[c] = read(c)
        for c in range(n_chunks):
            nxt = c + AHEAD + 1
            if nxt < n_chunks:
                s = nxt % NBUF
                if pending_writes[s] is not None:
                    for h in pending_writes[s]:
                        h.wait()
                    pending_writes[s] = None
                pending_reads[nxt] = read(nxt)
            pending_reads[c].wait()
            pending_writes[c % NBUF] = write(c)
        for s in range(NBUF):
            if pending_writes[s] is not None:
                for h in pending_writes[s]:
                    h.wait()

    scratch = (
        [pltpu.VMEM((CH, D), pos.dtype) for _ in range(NBUF)]
        + [pltpu.SemaphoreType.DMA for _ in range(NBUF + NBUF * batch_size)]
    )
    return pl.pallas_call(
        body,
        in_specs=[pl.BlockSpec(memory_space=pltpu.MemorySpace.HBM)],
        out_specs=pl.BlockSpec(memory_space=pltpu.MemorySpace.HBM),
        out_shape=jax.ShapeDtypeStruct((batch_size, S, D), pos.dtype),
        scratch_shapes=scratch,
    )(pos)


# speedup vs baseline: 1.0037x; 1.0037x over previous
import jax
import jax.numpy as jnp
from jax.experimental import pallas as pl
from jax.experimental.pallas import tpu as pltpu


def kernel(tokens, positional_embedding_weights):
    batch_size, seq_len = tokens.shape
    pos = positional_embedding_weights[:seq_len]
    S, D = pos.shape
    CH = 4096
    n_chunks = S // CH
    NBUF = 2
    AHEAD = 1

    def body(in_hbm, out_hbm, *refs):
        bufs = refs[:NBUF]
        rsems = refs[NBUF : 2 * NBUF]
        wsems = refs[2 * NBUF :]

        def read(c):
            s = c % NBUF
            cp = pltpu.make_async_copy(in_hbm.at[pl.ds(c * CH, CH)], bufs[s], rsems[s])
            cp.start()
            return cp

        def write(c):
            s = c % NBUF
            cps = []
            for b in range(batch_size):
                cp = pltpu.make_async_copy(
                    bufs[s], out_hbm.at[b, pl.ds(c * CH, CH)], wsems[s * batch_size + b]
                )
                cp.start()
                cps.append(cp)
            return cps

        pending_writes = [None] * NBUF
        pending_reads = [None] * n_chunks
        for c in range(min(AHEAD + 1, n_chunks)):
            pending_reads[c] = read(c)
        for c in range(n_chunks):
            nxt = c + AHEAD + 1
            if nxt < n_chunks:
                s = nxt % NBUF
                if pending_writes[s] is not None:
                    for h in pending_writes[s]:
                        h.wait()
                    pending_writes[s] = None
                pending_reads[nxt] = read(nxt)
            pending_reads[c].wait()
            pending_writes[c % NBUF] = write(c)
        for s in range(NBUF):
            if pending_writes[s] is not None:
                for h in pending_writes[s]:
                    h.wait()

    scratch = (
        [pltpu.VMEM((CH, D), pos.dtype) for _ in range(NBUF)]
        + [pltpu.SemaphoreType.DMA for _ in range(NBUF + NBUF * batch_size)]
    )
    return pl.pallas_call(
        body,
        in_specs=[pl.BlockSpec(memory_space=pltpu.MemorySpace.HBM)],
        out_specs=pl.BlockSpec(memory_space=pltpu.MemorySpace.HBM),
        out_shape=jax.ShapeDtypeStruct((batch_size, S, D), pos.dtype),
        scratch_shapes=scratch,
    )(pos)
